# baseline (speedup 1.0000x reference)
import functools

import jax
import jax.numpy as jnp
from jax import lax
from jax.experimental import pallas as pl
from jax.experimental.pallas import tpu as pltpu
from jax.experimental.pallas import tpu_sc as plsc

_B = 16384
_D = 64
_NC = 2
_NS = 16
_NW = _NC * _NS
_BPW = _B // _NW
_CHUNK = 128
_NCHUNK = _BPW // _CHUNK


@functools.partial(
    pl.kernel,
    out_type=jax.ShapeDtypeStruct((_B, _D), jnp.float32),
    mesh=plsc.VectorSubcoreMesh(core_axis_name="c", subcore_axis_name="s"),
    scratch_types=[
        pltpu.VMEM((_NCHUNK, _CHUNK), jnp.int32),
        pltpu.VMEM((_BPW, _D), jnp.float32),
        pltpu.SemaphoreType.DMA,
    ],
    compiler_params=pltpu.CompilerParams(use_tc_tiling_on_sc=False),
)
def _sc_gather(idx_hbm, table_hbm, out_hbm, idx_v, rows_v, sem):
    wid = lax.axis_index("s") * _NC + lax.axis_index("c")
    base = wid * _BPW
    pltpu.sync_copy(idx_hbm.at[wid], idx_v)
    copies = []
    for j in range(_NCHUNK):
        copies.append(
            pltpu.async_copy(
                table_hbm.at[idx_v.at[j]],
                rows_v.at[pl.ds(j * _CHUNK, _CHUNK), :],
                sem,
            )
        )
    for cp in copies:
        cp.wait()
    pltpu.sync_copy(rows_v, out_hbm.at[pl.ds(base, _BPW), :])


def kernel(token_ids, embedding):
    idx = token_ids.astype(jnp.int32).reshape(_NW, _NCHUNK, _CHUNK)
    return _sc_gather(idx, embedding)


# speedup vs baseline: 2.8133x; 2.8133x over previous
import functools

import jax
import jax.numpy as jnp
from jax import lax
from jax.experimental import pallas as pl
from jax.experimental.pallas import tpu as pltpu
from jax.experimental.pallas import tpu_sc as plsc

_B = 16384
_D = 64
_V = 1000000
_NC = 2
_NS = 16
_NW = _NC * _NS
_BPW = _B // _NW
_L = 128
_NBUF = 6
_G = _BPW // 16


@functools.partial(
    pl.kernel,
    out_type=jax.ShapeDtypeStruct((_B, _D), jnp.float32),
    mesh=plsc.VectorSubcoreMesh(core_axis_name="c", subcore_axis_name="s"),
    scratch_types=[
        pltpu.VMEM((_BPW,), jnp.int32),
        pltpu.VMEM((_NBUF, _D, _L), jnp.float32),
        pltpu.VMEM((_BPW, _D), jnp.float32),
        pltpu.SemaphoreType.DMA,
    ],
    compiler_params=pltpu.CompilerParams(
        use_tc_tiling_on_sc=True, needs_layout_passes=False
    ),
)
def _sc_gather(idx_hbm, tab_hbm, out_hbm, idx_v, blk_v, rows_v, sem):
    wid = lax.axis_index("s") * _NC + lax.axis_index("c")
    base = wid * _BPW
    pltpu.sync_copy(idx_hbm.at[pl.ds(base, _BPW)], idx_v)

    def _c0(r):
        return pl.multiple_of(jnp.minimum(r // _L, (_V - _L) // _L) * _L, _L)

    def _fetch(r, slot):
        pltpu.async_copy(tab_hbm.at[:, pl.ds(_c0(r), _L)], blk_v.at[slot], sem)

    def _drain(r, slot):
        pltpu.make_async_copy(
            tab_hbm.at[:, pl.ds(_c0(r), _L)], blk_v.at[slot], sem
        ).wait()

    def _extract(r, slot, j):
        lvec = jnp.full((16,), r - _c0(r), jnp.int32)
        blk = blk_v.at[slot]
        row = rows_v.at[j]
        for h in range(_D // 16):
            dvec = lax.iota(jnp.int32, 16) + (16 * h)
            row[pl.ds(16 * h, 16)] = plsc.load_gather(blk, [dvec, lvec])

    def body(g, carry):
        vec = idx_v[pl.ds(g * 16, 16)]
        for t in range(_NBUF):
            _fetch(vec[t], t)
        for t in range(16):
            _drain(vec[t], t % _NBUF)
            _extract(vec[t], t % _NBUF, g * 16 + t)
            if t + _NBUF < 16:
                _fetch(vec[t + _NBUF], t % _NBUF)
        return carry

    lax.fori_loop(0, _G, body, 0)
    pltpu.sync_copy(rows_v, out_hbm.at[pl.ds(base, _BPW), :])


def kernel(token_ids, embedding):
    return _sc_gather(token_ids.astype(jnp.int32), embedding.T)


# speedup vs baseline: 2.9902x; 1.0629x over previous
import functools

import jax
import jax.numpy as jnp
from jax import lax
from jax.experimental import pallas as pl
from jax.experimental.pallas import tpu as pltpu
from jax.experimental.pallas import tpu_sc as plsc

_B = 16384
_D = 64
_V = 1000000
_NC = 2
_NS = 16
_NW = _NC * _NS
_BPW = _B // _NW
_L = 128
_W = 8
_G = _BPW // 16
_HALF = _BPW // 2


@functools.partial(
    pl.kernel,
    out_type=jax.ShapeDtypeStruct((_B, _D), jnp.float32),
    mesh=plsc.VectorSubcoreMesh(core_axis_name="c", subcore_axis_name="s"),
    scratch_types=[
        pltpu.VMEM((_BPW + 16,), jnp.int32),
        pltpu.VMEM((_W, _D, _L), jnp.float32),
        pltpu.VMEM((_HALF, _D), jnp.float32),
        pltpu.SemaphoreType.DMA,
    ],
    compiler_params=pltpu.CompilerParams(
        use_tc_tiling_on_sc=True,
        needs_layout_passes=False,
        disable_bounds_checks=True,
    ),
)
def _sc_gather(idx_hbm, tab_hbm, out_hbm, idx_v, blk_v, rows_v, sem):
    wid = lax.axis_index("s") * _NC + lax.axis_index("c")
    base = wid * _BPW
    pltpu.sync_copy(idx_hbm.at[pl.ds(base, _BPW)], idx_v.at[pl.ds(0, _BPW)])
    pltpu.sync_copy(idx_hbm.at[pl.ds(base, 16)], idx_v.at[pl.ds(_BPW, 16)])

    def _c0(r):
        return pl.multiple_of((r // _L) * _L, _L)

    def _fetch(r, slot):
        pltpu.async_copy(tab_hbm.at[:, pl.ds(_c0(r), _L)], blk_v.at[slot], sem)

    def _drain(r, slot):
        pltpu.make_async_copy(
            tab_hbm.at[:, pl.ds(_c0(r), _L)], blk_v.at[slot], sem
        ).wait()

    def _extract(r, slot, j):
        lvec = jnp.full((16,), r - _c0(r), jnp.int32)
        blk = blk_v.at[slot]
        row = rows_v.at[j]
        for h in range(_D // 16):
            dvec = lax.iota(jnp.int32, 16) + (16 * h)
            row[pl.ds(16 * h, 16)] = plsc.load_gather(blk, [dvec, lvec])

    vec0 = idx_v[pl.ds(0, 16)]
    for t in range(_W):
        _fetch(vec0[t], t)

    def body(g, carry):
        cur = idx_v[pl.ds(g * 16, 16)]
        nxt = idx_v[pl.ds(g * 16 + 16, 16)]
        j0 = (g % (_G // 2)) * 16
        for t in range(16):
            slot = t % _W
            _drain(cur[t], slot)
            _extract(cur[t], slot, j0 + t)
            r_ahead = cur[t + _W] if t + _W < 16 else nxt[t + _W - 16]
            _fetch(r_ahead, slot)

        @pl.when(g % (_G // 2) == _G // 2 - 1)
        def _():
            half = g // (_G // 2)
            pltpu.sync_copy(
                rows_v, out_hbm.at[pl.ds(base + half * _HALF, _HALF), :]
            )

        return carry

    lax.fori_loop(0, _G, body, 0)

    tail = idx_v[pl.ds(_BPW, 16)]
    for t in range(_W):
        _drain(tail[t], t)


def kernel(token_ids, embedding):
    return _sc_gather(token_ids.astype(jnp.int32), embedding.T)


# speedup vs baseline: 3.2282x; 1.0796x over previous
import functools

import jax
import jax.numpy as jnp
from jax import lax
from jax.experimental import pallas as pl
from jax.experimental.pallas import tpu as pltpu
from jax.experimental.pallas import tpu_sc as plsc

_B = 16384
_D = 64
_V = 1000000
_NC = 2
_NS = 16
_NW = _NC * _NS
_BPW = _B // _NW
_L = 128
_W = 6
_K = _W - 2
_G = _BPW // 16
_CHUNK = 128
_NCH = _BPW // _CHUNK


@functools.partial(
    pl.kernel,
    out_type=jax.ShapeDtypeStruct((_B, _D), jnp.float32),
    mesh=plsc.VectorSubcoreMesh(core_axis_name="c", subcore_axis_name="s"),
    scratch_types=[
        pltpu.VMEM((_BPW + 16,), jnp.int32),
        pltpu.VMEM((_W, _D, _L), jnp.float32),
        pltpu.VMEM((_BPW, _D), jnp.float32),
        pltpu.SemaphoreType.DMA,
    ],
    compiler_params=pltpu.CompilerParams(
        use_tc_tiling_on_sc=True,
        needs_layout_passes=False,
        disable_bounds_checks=True,
    ),
)
def _sc_gather_sorted(idx_hbm, tab_hbm, out_hbm, idx_v, blk_v, rows_v, sem):
    wid = lax.axis_index("s") * _NC + lax.axis_index("c")
    base = wid * _BPW
    pltpu.sync_copy(idx_hbm.at[pl.ds(base, _BPW)], idx_v.at[pl.ds(0, _BPW)])
    pltpu.sync_copy(idx_hbm.at[pl.ds(base, 16)], idx_v.at[pl.ds(_BPW, 16)])

    def _c0_of(c):
        return pl.multiple_of(c * _L, _L)

    def _fetch(c, slot):
        pltpu.async_copy(
            tab_hbm.at[:, pl.ds(_c0_of(c), _L)], blk_v.at[slot], sem
        )

    def _drain_one():
        pltpu.make_async_copy(
            tab_hbm.at[:, pl.ds(0, _L)], blk_v.at[0], sem
        ).wait()

    def _extract(l, slot, j):
        lvec = jnp.full((16,), l, jnp.int32)
        blk = blk_v.at[slot]
        row = rows_v.at[j]
        for h in range(_D // 16):
            dvec = lax.iota(jnp.int32, 16) + (16 * h)
            row[pl.ds(16 * h, 16)] = plsc.load_gather(blk, [dvec, lvec])

    neg1 = jnp.int32(-1)

    vec0 = idx_v[pl.ds(0, 16)]
    dc_f = jnp.int32(0)
    prev_c_f = neg1
    for t in range(_K):
        c_t = vec0[t] // _L
        is_new = c_t != prev_c_f

        @pl.when(is_new)
        def _(c_t=c_t, dc_f=dc_f):
            _fetch(c_t, dc_f % _W)

        dc_f = dc_f + is_new.astype(jnp.int32)
        prev_c_f = c_t

    def body(g, carry):
        dc_f, prev_c_f, dc_d, prev_c_d = carry
        cur = idx_v[pl.ds(g * 16, 16)]
        nxt = idx_v[pl.ds(g * 16 + 16, 16)]
        for t in range(16):
            r = cur[t]
            c_t = r // _L
            is_new_d = c_t != prev_c_d

            @pl.when(is_new_d)
            def _():
                _drain_one()

            dc_d = dc_d + is_new_d.astype(jnp.int32)
            prev_c_d = c_t
            _extract(r - c_t * _L, (dc_d - 1) % _W, g * 16 + t)
            r_a = cur[t + _K] if t + _K < 16 else nxt[t + _K - 16]
            c_a = r_a // _L
            is_new_f = c_a != prev_c_f

            @pl.when(is_new_f)
            def _(c_a=c_a, dc_f=dc_f):
                _fetch(c_a, dc_f % _W)

            dc_f = dc_f + is_new_f.astype(jnp.int32)
            prev_c_f = c_a
        return dc_f, prev_c_f, dc_d, prev_c_d

    dc_f, _, dc_d, _ = lax.fori_loop(
        0, _G, body, (dc_f, prev_c_f, jnp.int32(0), neg1)
    )

    lax.fori_loop(0, dc_f - dc_d, lambda i, c: (_drain_one(), c)[1], 0)

    pltpu.sync_copy(rows_v, out_hbm.at[pl.ds(base, _BPW), :])


@functools.partial(
    pl.kernel,
    out_type=jax.ShapeDtypeStruct((_B, _D), jnp.float32),
    mesh=plsc.VectorSubcoreMesh(core_axis_name="c", subcore_axis_name="s"),
    scratch_types=[
        pltpu.VMEM((_NCH, _CHUNK), jnp.int32),
        pltpu.VMEM((_BPW, _D), jnp.float32),
        pltpu.SemaphoreType.DMA,
    ],
    compiler_params=pltpu.CompilerParams(use_tc_tiling_on_sc=False),
)
def _sc_scatter(pos_hbm, rows_hbm, out_hbm, pos_v, rows_v, sem):
    wid = lax.axis_index("s") * _NC + lax.axis_index("c")
    base = wid * _BPW
    pltpu.sync_copy(pos_hbm.at[wid], pos_v)
    pltpu.sync_copy(rows_hbm.at[pl.ds(base, _BPW), :], rows_v)
    copies = []
    for j in range(_NCH):
        copies.append(
            pltpu.async_copy(
                rows_v.at[pl.ds(j * _CHUNK, _CHUNK), :],
                out_hbm.at[pos_v.at[j]],
                sem,
            )
        )
    for cp in copies:
        cp.wait()


def kernel(token_ids, embedding):
    ids = token_ids.astype(jnp.int32)
    order = jnp.argsort(ids)
    ids_sorted = jnp.take(ids, order, axis=0)
    rows_sorted = _sc_gather_sorted(ids_sorted, embedding.T)
    pos = order.astype(jnp.int32).reshape(_NW, _NCH, _CHUNK)
    return _sc_scatter(pos, rows_sorted)


# speedup vs baseline: 4.0483x; 1.2540x over previous
import functools

import jax
import jax.numpy as jnp
from jax import lax
from jax.experimental import pallas as pl
from jax.experimental.pallas import tpu as pltpu
from jax.experimental.pallas import tpu_sc as plsc

_B = 16384
_D = 64
_V = 1000000
_NC = 2
_NS = 16
_NW = _NC * _NS
_BPW = _B // _NW
_L = 128
_W = 9
_K = _W - 2
_HALF = _BPW // 2
_G = _BPW // 16
_CHUNK = 128
_NCH = _BPW // _CHUNK


@functools.partial(
    pl.kernel,
    out_type=jax.ShapeDtypeStruct((_B, _D), jnp.float32),
    mesh=plsc.VectorSubcoreMesh(core_axis_name="c", subcore_axis_name="s"),
    scratch_types=[
        pltpu.VMEM((_BPW + 16,), jnp.int32),
        pltpu.VMEM((_W, _D, _L), jnp.float32),
        pltpu.VMEM((_HALF, _D), jnp.float32),
        pltpu.SemaphoreType.DMA,
    ],
    compiler_params=pltpu.CompilerParams(
        use_tc_tiling_on_sc=True,
        needs_layout_passes=False,
        disable_bounds_checks=True,
    ),
)
def _sc_gather_sorted(idx_hbm, tab_hbm, out_hbm, idx_v, blk_v, rows_v, sem):
    wid = lax.axis_index("s") * _NC + lax.axis_index("c")
    base = wid * _BPW
    pltpu.sync_copy(idx_hbm.at[pl.ds(base, _BPW)], idx_v.at[pl.ds(0, _BPW)])
    pltpu.sync_copy(idx_hbm.at[pl.ds(base, 16)], idx_v.at[pl.ds(_BPW, 16)])

    def _c0_of(c):
        return pl.multiple_of(c * _L, _L)

    def _fetch(c, slot):
        pltpu.async_copy(
            tab_hbm.at[:, pl.ds(_c0_of(c), _L)], blk_v.at[slot], sem
        )

    def _drain_one():
        pltpu.make_async_copy(
            tab_hbm.at[:, pl.ds(0, _L)], blk_v.at[0], sem
        ).wait()

    def _extract(l, slot, j):
        lvec = jnp.full((16,), l, jnp.int32)
        blk = blk_v.at[slot]
        row = rows_v.at[j]
        for h in range(_D // 16):
            dvec = lax.iota(jnp.int32, 16) + (16 * h)
            row[pl.ds(16 * h, 16)] = plsc.load_gather(blk, [dvec, lvec])

    neg1 = jnp.int32(-1)

    vec0 = idx_v[pl.ds(0, 16)]
    dc_f = jnp.int32(0)
    prev_c_f = neg1
    for t in range(_K):
        c_t = vec0[t] // _L
        is_new = c_t != prev_c_f

        @pl.when(is_new)
        def _(c_t=c_t, dc_f=dc_f):
            _fetch(c_t, dc_f % _W)

        dc_f = dc_f + is_new.astype(jnp.int32)
        prev_c_f = c_t

    def body(g, carry):
        dc_f, prev_c_f, dc_d, prev_c_d = carry
        cur = idx_v[pl.ds(g * 16, 16)]
        nxt = idx_v[pl.ds(g * 16 + 16, 16)]
        for t in range(16):
            r = cur[t]
            c_t = r // _L
            is_new_d = c_t != prev_c_d

            @pl.when(is_new_d)
            def _():
                _drain_one()

            dc_d = dc_d + is_new_d.astype(jnp.int32)
            prev_c_d = c_t
            _extract(r - c_t * _L, (dc_d - 1) % _W, (g % (_G // 2)) * 16 + t)
            r_a = cur[t + _K] if t + _K < 16 else nxt[t + _K - 16]
            c_a = r_a // _L
            is_new_f = c_a != prev_c_f

            @pl.when(is_new_f)
            def _(c_a=c_a, dc_f=dc_f):
                _fetch(c_a, dc_f % _W)

            dc_f = dc_f + is_new_f.astype(jnp.int32)
            prev_c_f = c_a

        @pl.when(g % (_G // 2) == _G // 2 - 1)
        def _():
            half = g // (_G // 2)
            pltpu.sync_copy(
                rows_v, out_hbm.at[pl.ds(base + half * _HALF, _HALF), :]
            )

        return dc_f, prev_c_f, dc_d, prev_c_d

    dc_f, _, dc_d, _ = lax.fori_loop(
        0, _G, body, (dc_f, prev_c_f, jnp.int32(0), neg1)
    )

    lax.fori_loop(0, dc_f - dc_d, lambda i, c: (_drain_one(), c)[1], 0)


@functools.partial(
    pl.kernel,
    out_type=jax.ShapeDtypeStruct((_B, _D), jnp.float32),
    mesh=plsc.VectorSubcoreMesh(core_axis_name="c", subcore_axis_name="s"),
    scratch_types=[
        pltpu.VMEM((_NCH, _CHUNK), jnp.int32),
        pltpu.VMEM((_BPW, _D), jnp.float32),
        pltpu.SemaphoreType.DMA,
    ],
    compiler_params=pltpu.CompilerParams(use_tc_tiling_on_sc=False),
)
def _sc_scatter(pos_hbm, rows_hbm, out_hbm, pos_v, rows_v, sem):
    wid = lax.axis_index("s") * _NC + lax.axis_index("c")
    base = wid * _BPW
    pltpu.sync_copy(pos_hbm.at[wid], pos_v)
    pltpu.sync_copy(rows_hbm.at[pl.ds(base, _BPW), :], rows_v)
    copies = []
    for j in range(_NCH):
        copies.append(
            pltpu.async_copy(
                rows_v.at[pl.ds(j * _CHUNK, _CHUNK), :],
                out_hbm.at[pos_v.at[j]],
                sem,
            )
        )
    for cp in copies:
        cp.wait()


def kernel(token_ids, embedding):
    ids = token_ids.astype(jnp.int32)
    order = jnp.argsort(ids)
    ids_sorted = jnp.take(ids, order, axis=0)
    rows_sorted = _sc_gather_sorted(ids_sorted, embedding.T)
    pos = order.astype(jnp.int32).reshape(_NW, _NCH, _CHUNK)
    return _sc_scatter(pos, rows_sorted)
